# baseline (speedup 1.0000x reference)
import numpy as np
import jax
import jax.numpy as jnp
from jax.experimental import pallas as pl
from jax.experimental.pallas import tpu as pltpu

L = 2048
H = 128
F = 4
K = L // 2
M = K // 2


def _build_dft_consts():
    t = np.arange(K, dtype=np.float64)
    m = np.arange(M, dtype=np.float64)
    ang_e = (2.0 * np.pi / K) * np.outer(m, t)
    ang_o = (2.0 * np.pi / L) * np.outer(2 * m + 1, t)
    altrow = np.where(np.arange(K) % 2 == 0, 1.0, -1.0)[None, :]
    cse = np.concatenate([np.cos(ang_e), -np.sin(ang_e), altrow], axis=0) / L
    cso = np.concatenate([np.cos(ang_o), -np.sin(ang_o)], axis=0) / L
    alpha_e = np.where(m == 0, 1.0, 2.0)[None, :]
    cic_e = (alpha_e * np.cos(ang_e.T))
    cis_e = (-alpha_e * np.sin(ang_e.T))
    cic_o = (2.0 * np.cos(ang_o.T))
    cis_o = (-2.0 * np.sin(ang_o.T))
    altcol = altrow.T.astype(np.float32)
    bf = lambda a: a.astype(np.float32).astype(jnp.bfloat16)
    return (bf(cse), bf(cso), bf(cic_e), bf(cis_e), bf(cic_o), bf(cis_o),
            altcol)


_CSE, _CSO, _CICE, _CISE, _CICO, _CISO, _ALTCOL = _build_dft_consts()


def _sigmoid(x):
    return 1.0 / (1.0 + jnp.exp(-x))


def _stg_body(x1_ref, x2_ref, x3_ref, x4_ref, cse_ref, cso_ref, cice_ref,
              cise_ref, cico_ref, ciso_ref, altc_ref, cwr_ref, cwi_ref,
              cwn_ref, w1_ref, b1_ref, w2t_ref, b2_ref, g1_ref, be1_ref,
              g2_ref, be2_ref, oa_ref, ob_ref):
    x1 = x1_ref[0]
    x2 = x2_ref[0]
    x3 = x3_ref[0]
    x4 = x4_ref[0]

    m3 = jnp.mean(x3, axis=0, keepdims=True)
    m4 = jnp.mean(x4, axis=0, keepdims=True)
    X = jnp.concatenate([m3, m4, m3, m4], axis=1)
    mu = jnp.mean(X, axis=1, keepdims=True)
    var = jnp.mean((X - mu) ** 2, axis=1, keepdims=True)
    Xn = (X - mu) * jax.lax.rsqrt(var + 1e-5) * g1_ref[...] + be1_ref[...]
    h = jnp.dot(Xn, w1_ref[...], preferred_element_type=jnp.float32)
    h = jnp.maximum(h + b1_ref[...], 0.0)
    logits = jnp.sum(h * w2t_ref[...], axis=1, keepdims=True) + b2_ref[...]
    lm = jnp.max(logits, axis=0, keepdims=True)
    e = jnp.exp(logits - lm)
    fw = e / jnp.sum(e, axis=0, keepdims=True)
    fw3 = fw.reshape(F, 1, 1)

    wr = _sigmoid(jnp.sum(fw3 * cwr_ref[...], axis=0))
    wi = _sigmoid(jnp.sum(fw3 * cwi_ref[...], axis=0))
    cwn = jnp.sum(fw3 * cwn_ref[...], axis=0)
    wrn = _sigmoid(cwn[0:1])

    u = jnp.concatenate([x1[:K] + x1[K:], x2[:K] + x2[K:]],
                        axis=1).astype(jnp.bfloat16)
    v = jnp.concatenate([x1[:K] - x1[K:], x2[:K] - x2[K:]],
                        axis=1).astype(jnp.bfloat16)
    ME = jnp.dot(cse_ref[...], u, preferred_element_type=jnp.float32)
    MO = jnp.dot(cso_ref[...], v, preferred_element_type=jnp.float32)

    def gate(re1, im1, re2, im2, wr_c, wi_c):
        wrb = 1.0 - wr_c
        fr = jnp.concatenate([re1 * wr_c - im1 * wi_c,
                              re2 * wrb + im2 * wi_c], axis=1)
        fi = jnp.concatenate([re1 * wi_c + im1 * wr_c,
                              im2 * wrb - re2 * wi_c], axis=1)
        return fr.astype(jnp.bfloat16), fi.astype(jnp.bfloat16)

    frE, fiE = gate(ME[:M, :H], ME[M:2 * M, :H], ME[:M, H:], ME[M:2 * M, H:],
                    wr[:M], wi[:M])
    frO, fiO = gate(MO[:M, :H], MO[M:, :H], MO[:M, H:], MO[M:, H:],
                    wr[M:], wi[M:])

    ren = ME[2 * M:2 * M + 1]
    frn = jnp.concatenate([ren[:, :H] * wrn, ren[:, H:] * (1.0 - wrn)],
                          axis=1)

    P = (jnp.dot(cice_ref[...], frE, preferred_element_type=jnp.float32)
         + jnp.dot(cise_ref[...], fiE, preferred_element_type=jnp.float32)
         + altc_ref[...] * frn)
    Q = (jnp.dot(cico_ref[...], frO, preferred_element_type=jnp.float32)
         + jnp.dot(ciso_ref[...], fiO, preferred_element_type=jnp.float32))

    g2 = g2_ref[...]
    be2 = be2_ref[...]

    def ln_store(y, x, ref, row0):
        s = y + x
        mu_ = jnp.mean(s, axis=1, keepdims=True)
        v_ = jnp.mean((s - mu_) ** 2, axis=1, keepdims=True)
        ref[0, row0:row0 + K, :] = ((s - mu_) * jax.lax.rsqrt(v_ + 1e-5)
                                    ) * g2 + be2

    top = P + Q
    bot = P - Q
    ln_store(top[:, :H], x1[:K], oa_ref, 0)
    ln_store(bot[:, :H], x1[K:], oa_ref, K)
    ln_store(top[:, H:], x2[:K], ob_ref, 0)
    ln_store(bot[:, H:], x2[K:], ob_ref, K)


def kernel(input_tensor1, input_tensor2, input_tensor3, input_tensor4,
           complex_weight, W1, b1, W2, b2, ln1_g, ln1_b, ln2_g, ln2_b):
    B = input_tensor1.shape[0]

    cw = jnp.transpose(complex_weight[0], (2, 3, 0, 1))
    cwr = jnp.concatenate([cw[:, 0, 0:K:2, :], cw[:, 0, 1:K:2, :]], axis=1)
    cwi = jnp.concatenate([cw[:, 1, 0:K:2, :], cw[:, 1, 1:K:2, :]], axis=1)
    cwn = cw[:, :, K, :]

    batch_in = lambda b: (b, 0, 0)
    const2 = lambda b: (0, 0)
    const3 = lambda b: (0, 0, 0)

    grid_spec = pl.GridSpec(
        grid=(B,),
        in_specs=[
            pl.BlockSpec((1, L, H), batch_in),
            pl.BlockSpec((1, L, H), batch_in),
            pl.BlockSpec((1, L, H // 2), batch_in),
            pl.BlockSpec((1, L, H // 2), batch_in),
            pl.BlockSpec((K + 1, K), const2),
            pl.BlockSpec((K, K), const2),
            pl.BlockSpec((K, M), const2),
            pl.BlockSpec((K, M), const2),
            pl.BlockSpec((K, M), const2),
            pl.BlockSpec((K, M), const2),
            pl.BlockSpec((K, 1), const2),
            pl.BlockSpec((F, K, H), const3),
            pl.BlockSpec((F, K, H), const3),
            pl.BlockSpec((F, 2, H), const3),
            pl.BlockSpec((2 * H, H), const2),
            pl.BlockSpec((1, H), const2),
            pl.BlockSpec((F, H), const2),
            pl.BlockSpec((F, 1), const2),
            pl.BlockSpec((1, 2 * H), const2),
            pl.BlockSpec((1, 2 * H), const2),
            pl.BlockSpec((1, H), const2),
            pl.BlockSpec((1, H), const2),
        ],
        out_specs=[
            pl.BlockSpec((1, L, H), batch_in),
            pl.BlockSpec((1, L, H), batch_in),
        ],
    )

    out_a, out_b = pl.pallas_call(
        _stg_body,
        grid_spec=grid_spec,
        out_shape=[
            jax.ShapeDtypeStruct((B, L, H), jnp.float32),
            jax.ShapeDtypeStruct((B, L, H), jnp.float32),
        ],
        compiler_params=pltpu.CompilerParams(
            dimension_semantics=("arbitrary",),
        ),
    )(
        input_tensor1, input_tensor2, input_tensor3, input_tensor4,
        jnp.asarray(_CSE), jnp.asarray(_CSO), jnp.asarray(_CICE),
        jnp.asarray(_CISE), jnp.asarray(_CICO), jnp.asarray(_CISO),
        jnp.asarray(_ALTCOL),
        cwr, cwi, cwn,
        W1, b1.reshape(1, H), W2.T, b2.reshape(F, 1),
        ln1_g.reshape(1, 2 * H), ln1_b.reshape(1, 2 * H),
        ln2_g.reshape(1, H), ln2_b.reshape(1, H),
    )
    return (out_a, out_b)


# speedup vs baseline: 1.0149x; 1.0149x over previous
import numpy as np
import jax
import jax.numpy as jnp
from jax.experimental import pallas as pl
from jax.experimental.pallas import tpu as pltpu

L = 2048
H = 128
F = 4
K = L // 2
M = K // 2


def _build_dft_consts():
    t = np.arange(K, dtype=np.float64)
    m = np.arange(M, dtype=np.float64)
    ang_e = (2.0 * np.pi / K) * np.outer(m, t)
    ang_o = (2.0 * np.pi / L) * np.outer(2 * m + 1, t)
    altrow = np.where(np.arange(K) % 2 == 0, 1.0, -1.0)[None, :]
    cse = np.concatenate([np.cos(ang_e), -np.sin(ang_e), altrow], axis=0) / L
    cso = np.concatenate([np.cos(ang_o), -np.sin(ang_o)], axis=0) / L
    alpha_e = np.where(m == 0, 1.0, 2.0)[None, :]
    cic_e = (alpha_e * np.cos(ang_e.T))
    cis_e = (-alpha_e * np.sin(ang_e.T))
    cic_o = (2.0 * np.cos(ang_o.T))
    cis_o = (-2.0 * np.sin(ang_o.T))
    altcol = altrow.T.astype(np.float32)
    bf = lambda a: a.astype(np.float32).astype(jnp.bfloat16)
    return (bf(cse), bf(cso), bf(cic_e), bf(cis_e), bf(cic_o), bf(cis_o),
            altcol)


_CSE, _CSO, _CICE, _CISE, _CICO, _CISO, _ALTCOL = _build_dft_consts()


def _sigmoid(x):
    return 1.0 / (1.0 + jnp.exp(-x))


def _stg_body(x1_ref, x2_ref, x3_ref, x4_ref, cse_ref, cso_ref, cice_ref,
              cise_ref, cico_ref, ciso_ref, altc_ref, cwp_ref,
              cwn_ref, w1_ref, b1_ref, w2t_ref, b2_ref, g1_ref, be1_ref,
              g2_ref, be2_ref, oa_ref, ob_ref):
    x1 = x1_ref[0]
    x2 = x2_ref[0]
    x3 = x3_ref[0]
    x4 = x4_ref[0]

    m3 = jnp.mean(x3, axis=0, keepdims=True)
    m4 = jnp.mean(x4, axis=0, keepdims=True)
    X = jnp.concatenate([m3, m4, m3, m4], axis=1)
    mu = jnp.mean(X, axis=1, keepdims=True)
    var = jnp.mean((X - mu) ** 2, axis=1, keepdims=True)
    Xn = (X - mu) * jax.lax.rsqrt(var + 1e-5) * g1_ref[...] + be1_ref[...]
    h = jnp.dot(Xn, w1_ref[...], preferred_element_type=jnp.float32)
    h = jnp.maximum(h + b1_ref[...], 0.0)
    logits = jnp.sum(h * w2t_ref[...], axis=1, keepdims=True) + b2_ref[...]
    lm = jnp.max(logits, axis=0, keepdims=True)
    e = jnp.exp(logits - lm)
    fw = e / jnp.sum(e, axis=0, keepdims=True)

    rmix = sum(fw[f:f + 1] * cwp_ref[2 * f] for f in range(F))
    imix = sum(fw[f:f + 1] * cwp_ref[2 * f + 1] for f in range(F))
    wr = _sigmoid(rmix)
    wi = _sigmoid(imix)
    wrn = _sigmoid(sum(fw[f:f + 1] * cwn_ref[2 * f] for f in range(F)))

    u = jnp.concatenate([x1[:K] + x1[K:], x2[:K] + x2[K:]],
                        axis=1).astype(jnp.bfloat16)
    v = jnp.concatenate([x1[:K] - x1[K:], x2[:K] - x2[K:]],
                        axis=1).astype(jnp.bfloat16)
    ME = jnp.dot(cse_ref[...], u, preferred_element_type=jnp.float32)
    MO = jnp.dot(cso_ref[...], v, preferred_element_type=jnp.float32)

    def gate(re1, im1, re2, im2, wr_c, wi_c):
        wrb = 1.0 - wr_c
        fr = jnp.concatenate([re1 * wr_c - im1 * wi_c,
                              re2 * wrb + im2 * wi_c], axis=1)
        fi = jnp.concatenate([re1 * wi_c + im1 * wr_c,
                              im2 * wrb - re2 * wi_c], axis=1)
        return fr.astype(jnp.bfloat16), fi.astype(jnp.bfloat16)

    frE, fiE = gate(ME[:M, :H], ME[M:2 * M, :H], ME[:M, H:], ME[M:2 * M, H:],
                    wr[:M], wi[:M])
    frO, fiO = gate(MO[:M, :H], MO[M:, :H], MO[:M, H:], MO[M:, H:],
                    wr[M:], wi[M:])

    ren = ME[2 * M:2 * M + 1]
    frn = jnp.concatenate([ren[:, :H] * wrn, ren[:, H:] * (1.0 - wrn)],
                          axis=1)

    P = (jnp.dot(cice_ref[...], frE, preferred_element_type=jnp.float32)
         + jnp.dot(cise_ref[...], fiE, preferred_element_type=jnp.float32)
         + altc_ref[...] * frn)
    Q = (jnp.dot(cico_ref[...], frO, preferred_element_type=jnp.float32)
         + jnp.dot(ciso_ref[...], fiO, preferred_element_type=jnp.float32))

    g2 = g2_ref[...]
    be2 = be2_ref[...]

    def ln_store(y, x, ref, row0):
        s = y + x
        mu_ = jnp.mean(s, axis=1, keepdims=True)
        v_ = jnp.mean((s - mu_) ** 2, axis=1, keepdims=True)
        ref[0, row0:row0 + K, :] = ((s - mu_) * jax.lax.rsqrt(v_ + 1e-5)
                                    ) * g2 + be2

    top = P + Q
    bot = P - Q
    ln_store(top[:, :H], x1[:K], oa_ref, 0)
    ln_store(bot[:, :H], x1[K:], oa_ref, K)
    ln_store(top[:, H:], x2[:K], ob_ref, 0)
    ln_store(bot[:, H:], x2[K:], ob_ref, K)


def kernel(input_tensor1, input_tensor2, input_tensor3, input_tensor4,
           complex_weight, W1, b1, W2, b2, ln1_g, ln1_b, ln2_g, ln2_b):
    B = input_tensor1.shape[0]

    cwp = jnp.transpose(complex_weight.reshape(K + 1, H, 2 * F), (2, 0, 1))
    cwm = cwp[:, :K].reshape(2 * F, M, 2, H)
    cwperm = jnp.concatenate([cwm[:, :, 0], cwm[:, :, 1]], axis=1)
    cwn = cwp[:, K]

    batch_in = lambda b: (b, 0, 0)
    const2 = lambda b: (0, 0)
    const3 = lambda b: (0, 0, 0)

    grid_spec = pl.GridSpec(
        grid=(B,),
        in_specs=[
            pl.BlockSpec((1, L, H), batch_in),
            pl.BlockSpec((1, L, H), batch_in),
            pl.BlockSpec((1, L, H // 2), batch_in),
            pl.BlockSpec((1, L, H // 2), batch_in),
            pl.BlockSpec((K + 1, K), const2),
            pl.BlockSpec((K, K), const2),
            pl.BlockSpec((K, M), const2),
            pl.BlockSpec((K, M), const2),
            pl.BlockSpec((K, M), const2),
            pl.BlockSpec((K, M), const2),
            pl.BlockSpec((K, 1), const2),
            pl.BlockSpec((2 * F, K, H), const3),
            pl.BlockSpec((2 * F, H), const2),
            pl.BlockSpec((2 * H, H), const2),
            pl.BlockSpec((1, H), const2),
            pl.BlockSpec((F, H), const2),
            pl.BlockSpec((F, 1), const2),
            pl.BlockSpec((1, 2 * H), const2),
            pl.BlockSpec((1, 2 * H), const2),
            pl.BlockSpec((1, H), const2),
            pl.BlockSpec((1, H), const2),
        ],
        out_specs=[
            pl.BlockSpec((1, L, H), batch_in),
            pl.BlockSpec((1, L, H), batch_in),
        ],
    )

    out_a, out_b = pl.pallas_call(
        _stg_body,
        grid_spec=grid_spec,
        out_shape=[
            jax.ShapeDtypeStruct((B, L, H), jnp.float32),
            jax.ShapeDtypeStruct((B, L, H), jnp.float32),
        ],
        compiler_params=pltpu.CompilerParams(
            dimension_semantics=("arbitrary",),
        ),
    )(
        input_tensor1, input_tensor2, input_tensor3, input_tensor4,
        jnp.asarray(_CSE), jnp.asarray(_CSO), jnp.asarray(_CICE),
        jnp.asarray(_CISE), jnp.asarray(_CICO), jnp.asarray(_CISO),
        jnp.asarray(_ALTCOL),
        cwperm, cwn,
        W1, b1.reshape(1, H), W2.T, b2.reshape(F, 1),
        ln1_g.reshape(1, 2 * H), ln1_b.reshape(1, 2 * H),
        ln2_g.reshape(1, H), ln2_b.reshape(1, H),
    )
    return (out_a, out_b)


# speedup vs baseline: 1.3998x; 1.3792x over previous
import numpy as np
import jax
import jax.numpy as jnp
from jax.experimental import pallas as pl
from jax.experimental.pallas import tpu as pltpu

L = 2048
H = 128
F = 4
K = L // 2
M = K // 2


def _build_dft_consts():
    t = np.arange(K, dtype=np.float64)
    m = np.arange(M, dtype=np.float64)
    ang_e = (2.0 * np.pi / K) * np.outer(m, t)
    ang_o = (2.0 * np.pi / L) * np.outer(2 * m + 1, t)
    altrow = np.where(np.arange(K) % 2 == 0, 1.0, -1.0)[None, :]
    cse = np.concatenate([np.cos(ang_e), -np.sin(ang_e), altrow], axis=0) / L
    cso = np.concatenate([np.cos(ang_o), -np.sin(ang_o)], axis=0) / L
    alpha_e = np.where(m == 0, 1.0, 2.0)[None, :]
    cic_e = (alpha_e * np.cos(ang_e.T))
    cis_e = (-alpha_e * np.sin(ang_e.T))
    cic_o = (2.0 * np.cos(ang_o.T))
    cis_o = (-2.0 * np.sin(ang_o.T))
    altcol = altrow.T.astype(np.float32)
    bf = lambda a: a.astype(np.float32).astype(jnp.bfloat16)
    return (bf(cse), bf(cso), bf(cic_e), bf(cis_e), bf(cic_o), bf(cis_o),
            altcol)


_CSE, _CSO, _CICE, _CISE, _CICO, _CISO, _ALTCOL = _build_dft_consts()


def _sigmoid(x):
    return 1.0 / (1.0 + jnp.exp(-x))


def _stg_body(x1_ref, x2_ref, x3_ref, x4_ref, cse_ref, cso_ref, cice_ref,
              cise_ref, cico_ref, ciso_ref, altc_ref, cwp_ref,
              cwn_ref, w1_ref, b1_ref, w2t_ref, b2_ref, g1_ref, be1_ref,
              g2_ref, be2_ref, oa_ref, ob_ref):
    x1 = x1_ref[0]
    x2 = x2_ref[0]
    x3 = x3_ref[0]
    x4 = x4_ref[0]

    m3 = jnp.mean(x3, axis=1, keepdims=True)
    m4 = jnp.mean(x4, axis=1, keepdims=True)
    X = jnp.concatenate([m3, m4, m3, m4], axis=0)
    mu = jnp.mean(X, axis=0, keepdims=True)
    var = jnp.mean((X - mu) ** 2, axis=0, keepdims=True)
    Xn = (X - mu) * jax.lax.rsqrt(var + 1e-5) * g1_ref[...] + be1_ref[...]
    h = jnp.dot(w1_ref[...], Xn, preferred_element_type=jnp.float32)
    h = jnp.maximum(h + b1_ref[...], 0.0)
    logits = jnp.dot(w2t_ref[...], h,
                     preferred_element_type=jnp.float32) + b2_ref[...]
    lm = jnp.max(logits, axis=0, keepdims=True)
    e = jnp.exp(logits - lm)
    fw = e / jnp.sum(e, axis=0, keepdims=True)

    rmix = sum(fw[f:f + 1] * cwp_ref[2 * f] for f in range(F))
    imix = sum(fw[f:f + 1] * cwp_ref[2 * f + 1] for f in range(F))
    wr = _sigmoid(rmix)
    wi = _sigmoid(imix)
    wrn = _sigmoid(sum(fw[f:f + 1] * cwn_ref[2 * f] for f in range(F)))

    u = jnp.concatenate([x1[:K] + x1[K:], x2[:K] + x2[K:]],
                        axis=1).astype(jnp.bfloat16)
    v = jnp.concatenate([x1[:K] - x1[K:], x2[:K] - x2[K:]],
                        axis=1).astype(jnp.bfloat16)
    ME = jnp.dot(cse_ref[...], u, preferred_element_type=jnp.float32)
    MO = jnp.dot(cso_ref[...], v, preferred_element_type=jnp.float32)

    def gate(re1, im1, re2, im2, wr_c, wi_c):
        wrb = 1.0 - wr_c
        fr = jnp.concatenate([re1 * wr_c - im1 * wi_c,
                              re2 * wrb + im2 * wi_c], axis=1)
        fi = jnp.concatenate([re1 * wi_c + im1 * wr_c,
                              im2 * wrb - re2 * wi_c], axis=1)
        return fr.astype(jnp.bfloat16), fi.astype(jnp.bfloat16)

    frE, fiE = gate(ME[:M, :H], ME[M:2 * M, :H], ME[:M, H:], ME[M:2 * M, H:],
                    wr[:M], wi[:M])
    frO, fiO = gate(MO[:M, :H], MO[M:, :H], MO[:M, H:], MO[M:, H:],
                    wr[M:], wi[M:])

    ren = ME[2 * M:2 * M + 1]
    frn = jnp.concatenate([ren[:, :H] * wrn, ren[:, H:] * (1.0 - wrn)],
                          axis=1)

    P = (jnp.dot(cice_ref[...], frE, preferred_element_type=jnp.float32)
         + jnp.dot(cise_ref[...], fiE, preferred_element_type=jnp.float32)
         + altc_ref[...] * frn)
    Q = (jnp.dot(cico_ref[...], frO, preferred_element_type=jnp.float32)
         + jnp.dot(ciso_ref[...], fiO, preferred_element_type=jnp.float32))

    g2 = g2_ref[...]
    be2 = be2_ref[...]

    def ln_store(y, x, ref, row0):
        s = y + x
        mu_ = jnp.mean(s, axis=1, keepdims=True)
        v_ = jnp.mean((s - mu_) ** 2, axis=1, keepdims=True)
        ref[0, row0:row0 + K, :] = ((s - mu_) * jax.lax.rsqrt(v_ + 1e-5)
                                    ) * g2 + be2

    top = P + Q
    bot = P - Q
    ln_store(top[:, :H], x1[:K], oa_ref, 0)
    ln_store(bot[:, :H], x1[K:], oa_ref, K)
    ln_store(top[:, H:], x2[:K], ob_ref, 0)
    ln_store(bot[:, H:], x2[K:], ob_ref, K)


def kernel(input_tensor1, input_tensor2, input_tensor3, input_tensor4,
           complex_weight, W1, b1, W2, b2, ln1_g, ln1_b, ln2_g, ln2_b):
    B = input_tensor1.shape[0]

    cwp = jnp.transpose(complex_weight.reshape(K + 1, H, 2 * F), (2, 0, 1))
    cwm = cwp[:, :K].reshape(2 * F, M, 2, H)
    cwperm = jnp.concatenate([cwm[:, :, 0], cwm[:, :, 1]], axis=1)
    cwn = cwp[:, K]

    batch_in = lambda b: (b, 0, 0)
    const2 = lambda b: (0, 0)
    const3 = lambda b: (0, 0, 0)

    grid_spec = pl.GridSpec(
        grid=(B,),
        in_specs=[
            pl.BlockSpec((1, L, H), batch_in),
            pl.BlockSpec((1, L, H), batch_in),
            pl.BlockSpec((1, H // 2, L), batch_in),
            pl.BlockSpec((1, H // 2, L), batch_in),
            pl.BlockSpec((K + 1, K), const2),
            pl.BlockSpec((K, K), const2),
            pl.BlockSpec((K, M), const2),
            pl.BlockSpec((K, M), const2),
            pl.BlockSpec((K, M), const2),
            pl.BlockSpec((K, M), const2),
            pl.BlockSpec((K, 1), const2),
            pl.BlockSpec((2 * F, K, H), const3),
            pl.BlockSpec((2 * F, H), const2),
            pl.BlockSpec((H, 2 * H), const2),
            pl.BlockSpec((H, 1), const2),
            pl.BlockSpec((F, H), const2),
            pl.BlockSpec((F, 1), const2),
            pl.BlockSpec((2 * H, 1), const2),
            pl.BlockSpec((2 * H, 1), const2),
            pl.BlockSpec((1, H), const2),
            pl.BlockSpec((1, H), const2),
        ],
        out_specs=[
            pl.BlockSpec((1, L, H), batch_in),
            pl.BlockSpec((1, L, H), batch_in),
        ],
    )

    out_a, out_b = pl.pallas_call(
        _stg_body,
        grid_spec=grid_spec,
        out_shape=[
            jax.ShapeDtypeStruct((B, L, H), jnp.float32),
            jax.ShapeDtypeStruct((B, L, H), jnp.float32),
        ],
        compiler_params=pltpu.CompilerParams(
            dimension_semantics=("arbitrary",),
        ),
    )(
        input_tensor1, input_tensor2,
        jnp.transpose(input_tensor3, (0, 2, 1)),
        jnp.transpose(input_tensor4, (0, 2, 1)),
        jnp.asarray(_CSE), jnp.asarray(_CSO), jnp.asarray(_CICE),
        jnp.asarray(_CISE), jnp.asarray(_CICO), jnp.asarray(_CISO),
        jnp.asarray(_ALTCOL),
        cwperm, cwn,
        W1.T, b1.reshape(H, 1), W2.T, b2.reshape(F, 1),
        ln1_g.reshape(2 * H, 1), ln1_b.reshape(2 * H, 1),
        ln2_g.reshape(1, H), ln2_b.reshape(1, H),
    )
    return (out_a, out_b)


# speedup vs baseline: 1.5493x; 1.1068x over previous
import numpy as np
import jax
import jax.numpy as jnp
from jax.experimental import pallas as pl
from jax.experimental.pallas import tpu as pltpu

L = 2048
H = 128
F = 4
K = L // 2
M = K // 2
NB = 2


def _build_dft_consts():
    t = np.arange(K, dtype=np.float64)
    m = np.arange(M, dtype=np.float64)
    ang_e = (2.0 * np.pi / K) * np.outer(m, t)
    ang_o = (2.0 * np.pi / L) * np.outer(2 * m + 1, t)
    altrow = np.where(np.arange(K) % 2 == 0, 1.0, -1.0)[None, :]
    cse = np.concatenate([np.cos(ang_e), -np.sin(ang_e), altrow], axis=0) / L
    cso = np.concatenate([np.cos(ang_o), -np.sin(ang_o)], axis=0) / L
    alpha_e = np.where(m == 0, 1.0, 2.0)[None, :]
    cic_e = (alpha_e * np.cos(ang_e.T))
    cis_e = (-alpha_e * np.sin(ang_e.T))
    cic_o = (2.0 * np.cos(ang_o.T))
    cis_o = (-2.0 * np.sin(ang_o.T))
    altcol = altrow.T.astype(np.float32)
    bf = lambda a: a.astype(np.float32).astype(jnp.bfloat16)
    return (bf(cse), bf(cso), bf(cic_e), bf(cis_e), bf(cic_o), bf(cis_o),
            altcol)


_CSE, _CSO, _CICE, _CISE, _CICO, _CISO, _ALTCOL = _build_dft_consts()


def _sigmoid(x):
    return 1.0 / (1.0 + jnp.exp(-x))


def _stg_body(x1_ref, x2_ref, x3_ref, x4_ref, cse_ref, cso_ref, cice_ref,
              cise_ref, cico_ref, ciso_ref, altc_ref, cwp_ref,
              cwn_ref, w1_ref, b1_ref, w2t_ref, b2_ref, g1_ref, be1_ref,
              g2_ref, be2_ref, oa_ref, ob_ref):
    for s in range(NB):
        _stg_one(s, x1_ref, x2_ref, x3_ref, x4_ref, cse_ref, cso_ref,
                 cice_ref, cise_ref, cico_ref, ciso_ref, altc_ref, cwp_ref,
                 cwn_ref, w1_ref, b1_ref, w2t_ref, b2_ref, g1_ref, be1_ref,
                 g2_ref, be2_ref, oa_ref, ob_ref)


def _stg_one(s, x1_ref, x2_ref, x3_ref, x4_ref, cse_ref, cso_ref, cice_ref,
             cise_ref, cico_ref, ciso_ref, altc_ref, cwp_ref,
             cwn_ref, w1_ref, b1_ref, w2t_ref, b2_ref, g1_ref, be1_ref,
             g2_ref, be2_ref, oa_ref, ob_ref):
    x1 = x1_ref[s]
    x2 = x2_ref[s]
    x3 = x3_ref[s]
    x4 = x4_ref[s]

    m3 = jnp.mean(x3, axis=1, keepdims=True)
    m4 = jnp.mean(x4, axis=1, keepdims=True)
    X = jnp.concatenate([m3, m4, m3, m4], axis=0)
    mu = jnp.mean(X, axis=0, keepdims=True)
    var = jnp.mean((X - mu) ** 2, axis=0, keepdims=True)
    Xn = (X - mu) * jax.lax.rsqrt(var + 1e-5) * g1_ref[...] + be1_ref[...]
    h = jnp.dot(w1_ref[...], Xn, preferred_element_type=jnp.float32)
    h = jnp.maximum(h + b1_ref[...], 0.0)
    logits = jnp.dot(w2t_ref[...], h,
                     preferred_element_type=jnp.float32) + b2_ref[...]
    lm = jnp.max(logits, axis=0, keepdims=True)
    e = jnp.exp(logits - lm)
    fw = e / jnp.sum(e, axis=0, keepdims=True)

    rmix = sum(fw[f:f + 1] * cwp_ref[2 * f] for f in range(F))
    imix = sum(fw[f:f + 1] * cwp_ref[2 * f + 1] for f in range(F))
    wr = _sigmoid(rmix)
    wi = _sigmoid(imix)
    wrn = _sigmoid(sum(fw[f:f + 1] * cwn_ref[2 * f] for f in range(F)))

    u = jnp.concatenate([x1[:K] + x1[K:], x2[:K] + x2[K:]],
                        axis=1).astype(jnp.bfloat16)
    v = jnp.concatenate([x1[:K] - x1[K:], x2[:K] - x2[K:]],
                        axis=1).astype(jnp.bfloat16)
    ME = jnp.dot(cse_ref[...], u, preferred_element_type=jnp.float32)
    MO = jnp.dot(cso_ref[...], v, preferred_element_type=jnp.float32)

    def gate(re1, im1, re2, im2, wr_c, wi_c):
        wrb = 1.0 - wr_c
        fr = jnp.concatenate([re1 * wr_c - im1 * wi_c,
                              re2 * wrb + im2 * wi_c], axis=1)
        fi = jnp.concatenate([re1 * wi_c + im1 * wr_c,
                              im2 * wrb - re2 * wi_c], axis=1)
        return fr.astype(jnp.bfloat16), fi.astype(jnp.bfloat16)

    frE, fiE = gate(ME[:M, :H], ME[M:2 * M, :H], ME[:M, H:], ME[M:2 * M, H:],
                    wr[:M], wi[:M])
    frO, fiO = gate(MO[:M, :H], MO[M:, :H], MO[:M, H:], MO[M:, H:],
                    wr[M:], wi[M:])

    ren = ME[2 * M:2 * M + 1]
    frn = jnp.concatenate([ren[:, :H] * wrn, ren[:, H:] * (1.0 - wrn)],
                          axis=1)

    P = (jnp.dot(cice_ref[...], frE, preferred_element_type=jnp.float32)
         + jnp.dot(cise_ref[...], fiE, preferred_element_type=jnp.float32)
         + altc_ref[...] * frn)
    Q = (jnp.dot(cico_ref[...], frO, preferred_element_type=jnp.float32)
         + jnp.dot(ciso_ref[...], fiO, preferred_element_type=jnp.float32))

    g2 = g2_ref[...]
    be2 = be2_ref[...]

    def ln_store(y, x, ref, row0):
        sv = y + x
        mu_ = jnp.mean(sv, axis=1, keepdims=True)
        v_ = jnp.mean((sv - mu_) ** 2, axis=1, keepdims=True)
        ref[s, row0:row0 + K, :] = ((sv - mu_) * jax.lax.rsqrt(v_ + 1e-5)
                                    ) * g2 + be2

    top = P + Q
    bot = P - Q
    ln_store(top[:, :H], x1[:K], oa_ref, 0)
    ln_store(bot[:, :H], x1[K:], oa_ref, K)
    ln_store(top[:, H:], x2[:K], ob_ref, 0)
    ln_store(bot[:, H:], x2[K:], ob_ref, K)


def kernel(input_tensor1, input_tensor2, input_tensor3, input_tensor4,
           complex_weight, W1, b1, W2, b2, ln1_g, ln1_b, ln2_g, ln2_b):
    B = input_tensor1.shape[0]

    cwp = jnp.transpose(complex_weight.reshape(K + 1, H, 2 * F), (2, 0, 1))
    cwm = cwp[:, :K].reshape(2 * F, M, 2, H)
    cwperm = jnp.concatenate([cwm[:, :, 0], cwm[:, :, 1]], axis=1)
    cwn = cwp[:, K]

    batch_in = lambda b: (b, 0, 0)
    const2 = lambda b: (0, 0)
    const3 = lambda b: (0, 0, 0)

    grid_spec = pl.GridSpec(
        grid=(B // NB,),
        in_specs=[
            pl.BlockSpec((NB, L, H), batch_in),
            pl.BlockSpec((NB, L, H), batch_in),
            pl.BlockSpec((NB, H // 2, L), batch_in),
            pl.BlockSpec((NB, H // 2, L), batch_in),
            pl.BlockSpec((K + 1, K), const2),
            pl.BlockSpec((K, K), const2),
            pl.BlockSpec((K, M), const2),
            pl.BlockSpec((K, M), const2),
            pl.BlockSpec((K, M), const2),
            pl.BlockSpec((K, M), const2),
            pl.BlockSpec((K, 1), const2),
            pl.BlockSpec((2 * F, K, H), const3),
            pl.BlockSpec((2 * F, H), const2),
            pl.BlockSpec((H, 2 * H), const2),
            pl.BlockSpec((H, 1), const2),
            pl.BlockSpec((F, H), const2),
            pl.BlockSpec((F, 1), const2),
            pl.BlockSpec((2 * H, 1), const2),
            pl.BlockSpec((2 * H, 1), const2),
            pl.BlockSpec((1, H), const2),
            pl.BlockSpec((1, H), const2),
        ],
        out_specs=[
            pl.BlockSpec((NB, L, H), batch_in),
            pl.BlockSpec((NB, L, H), batch_in),
        ],
    )

    out_a, out_b = pl.pallas_call(
        _stg_body,
        grid_spec=grid_spec,
        out_shape=[
            jax.ShapeDtypeStruct((B, L, H), jnp.float32),
            jax.ShapeDtypeStruct((B, L, H), jnp.float32),
        ],
        compiler_params=pltpu.CompilerParams(
            dimension_semantics=("arbitrary",),
        ),
    )(
        input_tensor1, input_tensor2,
        jnp.transpose(input_tensor3, (0, 2, 1)),
        jnp.transpose(input_tensor4, (0, 2, 1)),
        jnp.asarray(_CSE), jnp.asarray(_CSO), jnp.asarray(_CICE),
        jnp.asarray(_CISE), jnp.asarray(_CICO), jnp.asarray(_CISO),
        jnp.asarray(_ALTCOL),
        cwperm, cwn,
        W1.T, b1.reshape(H, 1), W2.T, b2.reshape(F, 1),
        ln1_g.reshape(2 * H, 1), ln1_b.reshape(2 * H, 1),
        ln2_g.reshape(1, H), ln2_b.reshape(1, H),
    )
    return (out_a, out_b)


# speedup vs baseline: 1.5671x; 1.0115x over previous
import numpy as np
import jax
import jax.numpy as jnp
from jax.experimental import pallas as pl
from jax.experimental.pallas import tpu as pltpu

L = 2048
H = 128
F = 4
K = L // 2
M = K // 2
NB = 2


def _build_dft_consts():
    t = np.arange(K, dtype=np.float64)
    m = np.arange(M, dtype=np.float64)
    ang_e = (2.0 * np.pi / K) * np.outer(m, t)
    ang_o = (2.0 * np.pi / L) * np.outer(2 * m + 1, t)
    altrow = np.where(np.arange(K) % 2 == 0, 1.0, -1.0)[None, :]
    cse = np.concatenate([np.cos(ang_e), -np.sin(ang_e), altrow], axis=0) / L
    cso = np.concatenate([np.cos(ang_o), -np.sin(ang_o)], axis=0) / L
    alpha_e = np.where(m == 0, 1.0, 2.0)[None, :]
    cic_e = (alpha_e * np.cos(ang_e.T))
    cis_e = (-alpha_e * np.sin(ang_e.T))
    cic_o = (2.0 * np.cos(ang_o.T))
    cis_o = (-2.0 * np.sin(ang_o.T))
    altcol = altrow.T.astype(np.float32)
    bf = lambda a: a.astype(np.float32).astype(jnp.bfloat16)
    return (bf(cse), bf(cso), bf(cic_e), bf(cis_e), bf(cic_o), bf(cis_o),
            altcol)


_CSE, _CSO, _CICE, _CISE, _CICO, _CISO, _ALTCOL = _build_dft_consts()


def _sigmoid(x):
    return 1.0 / (1.0 + jnp.exp(-x))


def _stg_body(x1_ref, x2_ref, x3_ref, x4_ref, cse_ref, cso_ref, cice_ref,
              cise_ref, cico_ref, ciso_ref, altc_ref, cwp_ref,
              cwn_ref, w1_ref, b1_ref, w2t_ref, b2_ref, g1_ref, be1_ref,
              g2_ref, be2_ref, oa_ref, ob_ref):
    for s in range(NB):
        _stg_one(s, x1_ref, x2_ref, x3_ref, x4_ref, cse_ref, cso_ref,
                 cice_ref, cise_ref, cico_ref, ciso_ref, altc_ref, cwp_ref,
                 cwn_ref, w1_ref, b1_ref, w2t_ref, b2_ref, g1_ref, be1_ref,
                 g2_ref, be2_ref, oa_ref, ob_ref)


def _stg_one(s, x1_ref, x2_ref, x3_ref, x4_ref, cse_ref, cso_ref, cice_ref,
             cise_ref, cico_ref, ciso_ref, altc_ref, cwp_ref,
             cwn_ref, w1_ref, b1_ref, w2t_ref, b2_ref, g1_ref, be1_ref,
             g2_ref, be2_ref, oa_ref, ob_ref):
    x1 = x1_ref[s]
    x2 = x2_ref[s]
    x3 = x3_ref[s]
    x4 = x4_ref[s]

    m3 = jnp.mean(x3, axis=1, keepdims=True)
    m4 = jnp.mean(x4, axis=1, keepdims=True)
    X = jnp.concatenate([m3, m4, m3, m4], axis=0)
    mu = jnp.mean(X, axis=0, keepdims=True)
    var = jnp.mean((X - mu) ** 2, axis=0, keepdims=True)
    Xn = (X - mu) * jax.lax.rsqrt(var + 1e-5) * g1_ref[...] + be1_ref[...]
    h = jnp.dot(w1_ref[...], Xn, preferred_element_type=jnp.float32)
    h = jnp.maximum(h + b1_ref[...], 0.0)
    logits = jnp.dot(w2t_ref[...], h,
                     preferred_element_type=jnp.float32) + b2_ref[...]
    lm = jnp.max(logits, axis=0, keepdims=True)
    e = jnp.exp(logits - lm)
    fw = e / jnp.sum(e, axis=0, keepdims=True)

    rmix = sum(fw[f:f + 1] * cwp_ref[2 * f] for f in range(F))
    imix = sum(fw[f:f + 1] * cwp_ref[2 * f + 1] for f in range(F))
    wr = _sigmoid(rmix)
    wi = _sigmoid(imix)
    wrn = _sigmoid(sum(fw[f:f + 1] * cwn_ref[2 * f] for f in range(F)))

    x1a, x1b = x1[:K].astype(jnp.bfloat16), x1[K:].astype(jnp.bfloat16)
    x2a, x2b = x2[:K].astype(jnp.bfloat16), x2[K:].astype(jnp.bfloat16)
    u = jnp.concatenate([x1a + x1b, x2a + x2b], axis=1)
    v = jnp.concatenate([x1a - x1b, x2a - x2b], axis=1)
    ME = jnp.dot(cse_ref[...], u, preferred_element_type=jnp.float32)
    MO = jnp.dot(cso_ref[...], v, preferred_element_type=jnp.float32)

    def gate(re1, im1, re2, im2, wr_c, wi_c):
        wrb = 1.0 - wr_c
        fr = jnp.concatenate([re1 * wr_c - im1 * wi_c,
                              re2 * wrb + im2 * wi_c], axis=1)
        fi = jnp.concatenate([re1 * wi_c + im1 * wr_c,
                              im2 * wrb - re2 * wi_c], axis=1)
        return fr, fi

    wr16 = wr.astype(jnp.bfloat16)
    wi16 = wi.astype(jnp.bfloat16)
    ME16 = ME.astype(jnp.bfloat16)
    MO16 = MO.astype(jnp.bfloat16)
    frE, fiE = gate(ME16[:M, :H], ME16[M:2 * M, :H], ME16[:M, H:],
                    ME16[M:2 * M, H:], wr16[:M], wi16[:M])
    frO, fiO = gate(MO16[:M, :H], MO16[M:, :H], MO16[:M, H:], MO16[M:, H:],
                    wr16[M:], wi16[M:])

    ren = ME[2 * M:2 * M + 1]
    frn = jnp.concatenate([ren[:, :H] * wrn, ren[:, H:] * (1.0 - wrn)],
                          axis=1)

    P = (jnp.dot(cice_ref[...], frE, preferred_element_type=jnp.float32)
         + jnp.dot(cise_ref[...], fiE, preferred_element_type=jnp.float32)
         + altc_ref[...] * frn)
    Q = (jnp.dot(cico_ref[...], frO, preferred_element_type=jnp.float32)
         + jnp.dot(ciso_ref[...], fiO, preferred_element_type=jnp.float32))

    g2 = g2_ref[...]
    be2 = be2_ref[...]

    def ln_store(y, x, ref, row0):
        sv = y + x
        mu_ = jnp.mean(sv, axis=1, keepdims=True)
        v_ = jnp.mean((sv - mu_) ** 2, axis=1, keepdims=True)
        ref[s, row0:row0 + K, :] = ((sv - mu_) * jax.lax.rsqrt(v_ + 1e-5)
                                    ) * g2 + be2

    top = P + Q
    bot = P - Q
    ln_store(top[:, :H], x1[:K], oa_ref, 0)
    ln_store(bot[:, :H], x1[K:], oa_ref, K)
    ln_store(top[:, H:], x2[:K], ob_ref, 0)
    ln_store(bot[:, H:], x2[K:], ob_ref, K)


def kernel(input_tensor1, input_tensor2, input_tensor3, input_tensor4,
           complex_weight, W1, b1, W2, b2, ln1_g, ln1_b, ln2_g, ln2_b):
    B = input_tensor1.shape[0]

    cwp = jnp.transpose(complex_weight.reshape(K + 1, H, 2 * F), (2, 0, 1))
    cwm = cwp[:, :K].reshape(2 * F, M, 2, H)
    cwperm = jnp.concatenate([cwm[:, :, 0], cwm[:, :, 1]], axis=1)
    cwn = cwp[:, K]

    batch_in = lambda b: (b, 0, 0)
    const2 = lambda b: (0, 0)
    const3 = lambda b: (0, 0, 0)

    grid_spec = pl.GridSpec(
        grid=(B // NB,),
        in_specs=[
            pl.BlockSpec((NB, L, H), batch_in),
            pl.BlockSpec((NB, L, H), batch_in),
            pl.BlockSpec((NB, H // 2, L), batch_in),
            pl.BlockSpec((NB, H // 2, L), batch_in),
            pl.BlockSpec((K + 1, K), const2),
            pl.BlockSpec((K, K), const2),
            pl.BlockSpec((K, M), const2),
            pl.BlockSpec((K, M), const2),
            pl.BlockSpec((K, M), const2),
            pl.BlockSpec((K, M), const2),
            pl.BlockSpec((K, 1), const2),
            pl.BlockSpec((2 * F, K, H), const3),
            pl.BlockSpec((2 * F, H), const2),
            pl.BlockSpec((H, 2 * H), const2),
            pl.BlockSpec((H, 1), const2),
            pl.BlockSpec((F, H), const2),
            pl.BlockSpec((F, 1), const2),
            pl.BlockSpec((2 * H, 1), const2),
            pl.BlockSpec((2 * H, 1), const2),
            pl.BlockSpec((1, H), const2),
            pl.BlockSpec((1, H), const2),
        ],
        out_specs=[
            pl.BlockSpec((NB, L, H), batch_in),
            pl.BlockSpec((NB, L, H), batch_in),
        ],
    )

    out_a, out_b = pl.pallas_call(
        _stg_body,
        grid_spec=grid_spec,
        out_shape=[
            jax.ShapeDtypeStruct((B, L, H), jnp.float32),
            jax.ShapeDtypeStruct((B, L, H), jnp.float32),
        ],
        compiler_params=pltpu.CompilerParams(
            dimension_semantics=("arbitrary",),
        ),
    )(
        input_tensor1, input_tensor2,
        jnp.transpose(input_tensor3, (0, 2, 1)),
        jnp.transpose(input_tensor4, (0, 2, 1)),
        jnp.asarray(_CSE), jnp.asarray(_CSO), jnp.asarray(_CICE),
        jnp.asarray(_CISE), jnp.asarray(_CICO), jnp.asarray(_CISO),
        jnp.asarray(_ALTCOL),
        cwperm, cwn,
        W1.T, b1.reshape(H, 1), W2.T, b2.reshape(F, 1),
        ln1_g.reshape(2 * H, 1), ln1_b.reshape(2 * H, 1),
        ln2_g.reshape(1, H), ln2_b.reshape(1, H),
    )
    return (out_a, out_b)


# speedup vs baseline: 1.6537x; 1.0553x over previous
import numpy as np
import jax
import jax.numpy as jnp
from jax.experimental import pallas as pl
from jax.experimental.pallas import tpu as pltpu

L = 2048
H = 128
F = 4
K = L // 2
M = K // 2
NB = 4


def _build_dft_consts():
    t = np.arange(K, dtype=np.float64)
    m = np.arange(M, dtype=np.float64)
    ang_e = (2.0 * np.pi / K) * np.outer(m, t)
    ang_o = (2.0 * np.pi / L) * np.outer(2 * m + 1, t)
    altrow = np.where(np.arange(K) % 2 == 0, 1.0, -1.0)[None, :]
    cse = np.concatenate([np.cos(ang_e), -np.sin(ang_e), altrow], axis=0) / L
    cso = np.concatenate([np.cos(ang_o), -np.sin(ang_o)], axis=0) / L
    alpha_e = np.where(m == 0, 1.0, 2.0)[None, :]
    cic_e = (alpha_e * np.cos(ang_e.T))
    cis_e = (-alpha_e * np.sin(ang_e.T))
    cic_o = (2.0 * np.cos(ang_o.T))
    cis_o = (-2.0 * np.sin(ang_o.T))
    altcol = altrow.T.astype(np.float32)
    bf = lambda a: a.astype(np.float32).astype(jnp.bfloat16)
    return (bf(cse), bf(cso), bf(cic_e), bf(cis_e), bf(cic_o), bf(cis_o),
            altcol)


_CSE, _CSO, _CICE, _CISE, _CICO, _CISO, _ALTCOL = _build_dft_consts()


def _sigmoid(x):
    return 1.0 / (1.0 + jnp.exp(-x))


def _stg_body(x1_ref, x2_ref, x3_ref, x4_ref, cse_ref, cso_ref, cice_ref,
              cise_ref, cico_ref, ciso_ref, altc_ref, cwp_ref,
              cwn_ref, w1_ref, b1_ref, w2t_ref, b2_ref, g1_ref, be1_ref,
              g2_ref, be2_ref, oa_ref, ob_ref):
    for s in range(NB):
        _stg_one(s, x1_ref, x2_ref, x3_ref, x4_ref, cse_ref, cso_ref,
                 cice_ref, cise_ref, cico_ref, ciso_ref, altc_ref, cwp_ref,
                 cwn_ref, w1_ref, b1_ref, w2t_ref, b2_ref, g1_ref, be1_ref,
                 g2_ref, be2_ref, oa_ref, ob_ref)


def _stg_one(s, x1_ref, x2_ref, x3_ref, x4_ref, cse_ref, cso_ref, cice_ref,
             cise_ref, cico_ref, ciso_ref, altc_ref, cwp_ref,
             cwn_ref, w1_ref, b1_ref, w2t_ref, b2_ref, g1_ref, be1_ref,
             g2_ref, be2_ref, oa_ref, ob_ref):
    x1 = x1_ref[s]
    x2 = x2_ref[s]
    x3 = x3_ref[s]
    x4 = x4_ref[s]

    m3 = jnp.mean(x3, axis=1, keepdims=True)
    m4 = jnp.mean(x4, axis=1, keepdims=True)
    X = jnp.concatenate([m3, m4, m3, m4], axis=0)
    mu = jnp.mean(X, axis=0, keepdims=True)
    var = jnp.mean((X - mu) ** 2, axis=0, keepdims=True)
    Xn = (X - mu) * jax.lax.rsqrt(var + 1e-5) * g1_ref[...] + be1_ref[...]
    h = jnp.dot(w1_ref[...], Xn, preferred_element_type=jnp.float32)
    h = jnp.maximum(h + b1_ref[...], 0.0)
    logits = jnp.dot(w2t_ref[...], h,
                     preferred_element_type=jnp.float32) + b2_ref[...]
    lm = jnp.max(logits, axis=0, keepdims=True)
    e = jnp.exp(logits - lm)
    fw = e / jnp.sum(e, axis=0, keepdims=True)

    rmix = sum(fw[f:f + 1] * cwp_ref[2 * f] for f in range(F))
    imix = sum(fw[f:f + 1] * cwp_ref[2 * f + 1] for f in range(F))
    wr = _sigmoid(rmix)
    wi = _sigmoid(imix)
    wrn = _sigmoid(sum(fw[f:f + 1] * cwn_ref[2 * f] for f in range(F)))

    x1a, x1b = x1[:K].astype(jnp.bfloat16), x1[K:].astype(jnp.bfloat16)
    x2a, x2b = x2[:K].astype(jnp.bfloat16), x2[K:].astype(jnp.bfloat16)
    u = jnp.concatenate([x1a + x1b, x2a + x2b], axis=1)
    v = jnp.concatenate([x1a - x1b, x2a - x2b], axis=1)
    ME = jnp.dot(cse_ref[...], u, preferred_element_type=jnp.float32)
    MO = jnp.dot(cso_ref[...], v, preferred_element_type=jnp.float32)

    def gate(re1, im1, re2, im2, wr_c, wi_c):
        wrb = 1.0 - wr_c
        fr = jnp.concatenate([re1 * wr_c - im1 * wi_c,
                              re2 * wrb + im2 * wi_c], axis=1)
        fi = jnp.concatenate([re1 * wi_c + im1 * wr_c,
                              im2 * wrb - re2 * wi_c], axis=1)
        return fr, fi

    wr16 = wr.astype(jnp.bfloat16)
    wi16 = wi.astype(jnp.bfloat16)
    ME16 = ME.astype(jnp.bfloat16)
    MO16 = MO.astype(jnp.bfloat16)
    frE, fiE = gate(ME16[:M, :H], ME16[M:2 * M, :H], ME16[:M, H:],
                    ME16[M:2 * M, H:], wr16[:M], wi16[:M])
    frO, fiO = gate(MO16[:M, :H], MO16[M:, :H], MO16[:M, H:], MO16[M:, H:],
                    wr16[M:], wi16[M:])

    ren = ME[2 * M:2 * M + 1]
    frn = jnp.concatenate([ren[:, :H] * wrn, ren[:, H:] * (1.0 - wrn)],
                          axis=1)

    P = (jnp.dot(cice_ref[...], frE, preferred_element_type=jnp.float32)
         + jnp.dot(cise_ref[...], fiE, preferred_element_type=jnp.float32)
         + altc_ref[...] * frn)
    Q = (jnp.dot(cico_ref[...], frO, preferred_element_type=jnp.float32)
         + jnp.dot(ciso_ref[...], fiO, preferred_element_type=jnp.float32))

    g2 = g2_ref[...]
    be2 = be2_ref[...]

    def ln_store(y, x, ref, row0):
        sv = y + x
        mu_ = jnp.mean(sv, axis=1, keepdims=True)
        v_ = jnp.mean((sv - mu_) ** 2, axis=1, keepdims=True)
        ref[s, row0:row0 + K, :] = ((sv - mu_) * jax.lax.rsqrt(v_ + 1e-5)
                                    ) * g2 + be2

    top = P + Q
    bot = P - Q
    ln_store(top[:, :H], x1[:K], oa_ref, 0)
    ln_store(bot[:, :H], x1[K:], oa_ref, K)
    ln_store(top[:, H:], x2[:K], ob_ref, 0)
    ln_store(bot[:, H:], x2[K:], ob_ref, K)


def kernel(input_tensor1, input_tensor2, input_tensor3, input_tensor4,
           complex_weight, W1, b1, W2, b2, ln1_g, ln1_b, ln2_g, ln2_b):
    B = input_tensor1.shape[0]

    cwp = jnp.transpose(complex_weight.reshape(K + 1, H, 2 * F), (2, 0, 1))
    cwm = cwp[:, :K].reshape(2 * F, M, 2, H)
    cwperm = jnp.concatenate([cwm[:, :, 0], cwm[:, :, 1]], axis=1)
    cwn = cwp[:, K]

    batch_in = lambda b: (b, 0, 0)
    const2 = lambda b: (0, 0)
    const3 = lambda b: (0, 0, 0)

    grid_spec = pl.GridSpec(
        grid=(B // NB,),
        in_specs=[
            pl.BlockSpec((NB, L, H), batch_in),
            pl.BlockSpec((NB, L, H), batch_in),
            pl.BlockSpec((NB, H // 2, L), batch_in),
            pl.BlockSpec((NB, H // 2, L), batch_in),
            pl.BlockSpec((K + 1, K), const2),
            pl.BlockSpec((K, K), const2),
            pl.BlockSpec((K, M), const2),
            pl.BlockSpec((K, M), const2),
            pl.BlockSpec((K, M), const2),
            pl.BlockSpec((K, M), const2),
            pl.BlockSpec((K, 1), const2),
            pl.BlockSpec((2 * F, K, H), const3),
            pl.BlockSpec((2 * F, H), const2),
            pl.BlockSpec((H, 2 * H), const2),
            pl.BlockSpec((H, 1), const2),
            pl.BlockSpec((F, H), const2),
            pl.BlockSpec((F, 1), const2),
            pl.BlockSpec((2 * H, 1), const2),
            pl.BlockSpec((2 * H, 1), const2),
            pl.BlockSpec((1, H), const2),
            pl.BlockSpec((1, H), const2),
        ],
        out_specs=[
            pl.BlockSpec((NB, L, H), batch_in),
            pl.BlockSpec((NB, L, H), batch_in),
        ],
    )

    out_a, out_b = pl.pallas_call(
        _stg_body,
        grid_spec=grid_spec,
        out_shape=[
            jax.ShapeDtypeStruct((B, L, H), jnp.float32),
            jax.ShapeDtypeStruct((B, L, H), jnp.float32),
        ],
        compiler_params=pltpu.CompilerParams(
            dimension_semantics=("arbitrary",),
        ),
    )(
        input_tensor1, input_tensor2,
        jnp.transpose(input_tensor3, (0, 2, 1)),
        jnp.transpose(input_tensor4, (0, 2, 1)),
        jnp.asarray(_CSE), jnp.asarray(_CSO), jnp.asarray(_CICE),
        jnp.asarray(_CISE), jnp.asarray(_CICO), jnp.asarray(_CISO),
        jnp.asarray(_ALTCOL),
        cwperm, cwn,
        W1.T, b1.reshape(H, 1), W2.T, b2.reshape(F, 1),
        ln1_g.reshape(2 * H, 1), ln1_b.reshape(2 * H, 1),
        ln2_g.reshape(1, H), ln2_b.reshape(1, H),
    )
    return (out_a, out_b)
